# baseline (speedup 1.0000x reference)
import functools

import jax
import jax.numpy as jnp
from jax import lax
from jax.experimental import pallas as pl
from jax.experimental.pallas import tpu as pltpu
from jax.experimental.pallas import tpu_sc as plsc

NUM_E = 1000000
H_DIM = 64
BATCH = 16384

_NC = 2
_NS = 16
_L = 16
_NW = _NC * _NS
_NB = (NUM_E + 127) // 128
_BPW = (_NB + _NW - 1) // _NW
_CAP = 768
_CHUNK = 2048
_TRASH = BATCH

_mesh = plsc.VectorSubcoreMesh(core_axis_name="c", subcore_axis_name="s")


def _vscalar(ref, i):
    v = plsc.load_gather(ref, [jnp.full((_L,), 0, jnp.int32) + i])
    return jax.lax.reduce_max(v, (0,))


@functools.partial(
    pl.kernel,
    mesh=_mesh,
    out_type=jax.ShapeDtypeStruct((BATCH + 1, 2 * H_DIM), jnp.float32),
    scratch_types=[
        pltpu.VMEM((_CHUNK,), jnp.int32),
        pltpu.VMEM((_CAP,), jnp.int32),
        pltpu.VMEM((_CAP,), jnp.int32),
        pltpu.VMEM((_CAP,), jnp.int32),
        pltpu.VMEM((_CAP // 128, 128), jnp.int32),
        pltpu.VMEM((256,), jnp.int32),
        pltpu.VMEM((256,), jnp.int32),
        pltpu.VMEM((H_DIM, 128), jnp.float32),
        pltpu.VMEM((_CAP, 2 * H_DIM), jnp.float32),
        pltpu.SemaphoreType.DMA,
    ],
    compiler_params=pltpu.CompilerParams(needs_layout_passes=False),
)
def _gate_sc(table_hbm, tail_hbm, idx_hbm, out_hbm,
             chunk_v, hits_j, hits_p, lane_v, pos_v, mark_v, blist_v,
             blk_v, out_v, sem):
    wid = lax.axis_index("s") * _NC + lax.axis_index("c")
    lo = wid * _BPW
    hi = jnp.minimum(lo + _BPW, _NB)
    iota = lax.iota(jnp.int32, _L)
    ones = jnp.full((_L,), 1, jnp.int32)

    for k in range(_CAP // _L):
        hits_j[pl.ds(k * _L, _L)] = ones * -1
    for q in range(_CAP // 128):
        for g in range(128 // _L):
            pos_v[q, pl.ds(g * _L, _L)] = ones * _TRASH
    for k in range(256 // _L):
        mark_v[pl.ds(k * _L, _L)] = ones * 0

    def scan_chunk(c, hcnt):
        pltpu.sync_copy(idx_hbm.at[pl.ds(c * _CHUNK, _CHUNK)], chunk_v)

        def scan_vec(k, hcnt):
            jv = chunk_v[pl.ds(k * _L, _L)]
            bbv = lax.shift_right_logical(jv, 7)
            m = (bbv >= lo) & (bbv < hi)
            mi = m.astype(jnp.int32)
            rank = plsc.cumsum(mi) - mi
            slot = hcnt + rank
            plsc.store_scatter(hits_j, [slot], jv, mask=m)
            plsc.store_scatter(hits_p, [slot], c * _CHUNK + k * _L + iota,
                               mask=m)
            plsc.store_scatter(mark_v, [bbv - lo], ones, mask=m)
            return hcnt + jnp.sum(mi)

        return lax.fori_loop(0, _CHUNK // _L, scan_vec, hcnt)

    hcnt = lax.fori_loop(0, BATCH // _CHUNK, scan_chunk, 0)

    mcnt = 0
    for k in range(256 // _L):
        mk = mark_v[pl.ds(k * _L, _L)]
        m = mk > 0
        mi = m.astype(jnp.int32)
        rank = plsc.cumsum(mi) - mi
        plsc.store_scatter(blist_v, [mcnt + rank], iota + (k * _L + lo),
                           mask=m)
        mcnt = mcnt + jnp.sum(mi)

    nvec = lax.div(hcnt + _L - 1, _L)

    def do_block(mth, outcnt):
        bb = _vscalar(blist_v, mth)

        @pl.when(bb == _NB - 1)
        def _():
            pltpu.sync_copy(tail_hbm, blk_v)

        @pl.when(bb != _NB - 1)
        def _():
            off = pl.multiple_of(bb * 128, 128)
            pltpu.sync_copy(table_hbm.at[:, pl.ds(off, 128)], blk_v)

        def rescan_vec(v, nb):
            ids = v * _L + iota
            jv = plsc.load_gather(hits_j, [ids])
            pv = plsc.load_gather(hits_p, [ids])
            m = lax.shift_right_logical(jv, 7) == bb
            mi = m.astype(jnp.int32)
            rank = plsc.cumsum(mi) - mi
            slot = outcnt + nb + rank
            plsc.store_scatter(lane_v, [slot], jv & 127, mask=m)
            plsc.store_scatter(
                pos_v, [lax.shift_right_logical(slot, 7), slot & 127], pv,
                mask=m)
            return nb + jnp.sum(mi)

        nb = lax.fori_loop(0, nvec, rescan_vec, 0)

        def extract(h, carry):
            lane = plsc.load_gather(lane_v, [jnp.full((_L,), 0, jnp.int32) + h])
            for g in range(H_DIM // _L):
                x = plsc.load_gather(blk_v, [iota + g * _L, lane])
                out_v[h, pl.ds(g * _L, _L)] = 1.0 / (1.0 + jnp.exp(-x))
            return carry

        lax.fori_loop(outcnt, outcnt + nb, extract, 0)
        return outcnt + nb

    lax.fori_loop(0, mcnt, do_block, 0)

    copies = []
    for q in range(_CAP // 128):
        copies.append(pltpu.async_copy(
            out_v.at[pl.ds(q * 128, 128)], out_hbm.at[pos_v.at[q]], sem))
    for c in copies:
        c.wait()


def kernel(X, Y, gate_theta):
    del X
    table_t = gate_theta.T
    tail = jnp.pad(gate_theta[(_NB - 1) * 128:].T,
                   ((0, 0), (0, _NB * 128 - NUM_E)))
    out2 = _gate_sc(table_t, tail, Y.astype(jnp.int32))
    return out2[:BATCH, :H_DIM]


# speedup vs baseline: 1.8581x; 1.8581x over previous
import functools

import jax
import jax.numpy as jnp
from jax import lax
from jax.experimental import pallas as pl
from jax.experimental.pallas import tpu as pltpu
from jax.experimental.pallas import tpu_sc as plsc

NUM_E = 1000000
H_DIM = 64
BATCH = 16384

_NC = 2
_NS = 16
_L = 16
_NW = _NC * _NS
_NB = (NUM_E + 127) // 128
_BPW = (_NB + _NW - 1) // _NW
_CAP = 640
_CHUNK = 2048
_TRASH = BATCH

_mesh = plsc.VectorSubcoreMesh(core_axis_name="c", subcore_axis_name="s")


def _vscalar(ref, i):
    v = plsc.load_gather(ref, [jnp.full((_L,), 0, jnp.int32) + i])
    return jax.lax.reduce_max(v, (0,))


@functools.partial(
    pl.kernel,
    mesh=_mesh,
    out_type=jax.ShapeDtypeStruct((BATCH + 1, 2 * H_DIM), jnp.float32),
    scratch_types=[
        pltpu.VMEM((_CHUNK,), jnp.int32),
        pltpu.VMEM((_CAP,), jnp.int32),
        pltpu.VMEM((_CAP,), jnp.int32),
        pltpu.VMEM((_CAP,), jnp.int32),
        pltpu.VMEM((_CAP // 128, 128), jnp.int32),
        pltpu.VMEM((256,), jnp.int32),
        pltpu.VMEM((256,), jnp.int32),
        pltpu.VMEM((H_DIM, 128), jnp.float32),
        pltpu.VMEM((H_DIM, 128), jnp.float32),
        pltpu.VMEM((H_DIM, 128), jnp.float32),
        pltpu.VMEM((H_DIM, 128), jnp.float32),
        pltpu.VMEM((_CAP, 2 * H_DIM), jnp.float32),
        pltpu.SemaphoreType.DMA,
        pltpu.SemaphoreType.DMA,
        pltpu.SemaphoreType.DMA,
        pltpu.SemaphoreType.DMA,
        pltpu.SemaphoreType.DMA,
    ],
    compiler_params=pltpu.CompilerParams(needs_layout_passes=False),
)
def _gate_sc(table_hbm, tail_hbm, idx_hbm, out_hbm,
             chunk_v, hits_j, hits_p, lane_v, pos_v, mark_v, blist_v,
             blk0_v, blk1_v, blk2_v, blk3_v, out_v,
             sem0, sem1, sem2, sem3, sem_out):
    wid = lax.axis_index("s") * _NC + lax.axis_index("c")
    lo = wid * _BPW
    hi = jnp.minimum(lo + _BPW, _NB)
    iota = lax.iota(jnp.int32, _L)
    ones = jnp.full((_L,), 1, jnp.int32)

    for k in range(_CAP // _L):
        hits_j[pl.ds(k * _L, _L)] = ones * -1
    for q in range(_CAP // 128):
        for g in range(128 // _L):
            pos_v[q, pl.ds(g * _L, _L)] = ones * _TRASH
    for k in range(256 // _L):
        mark_v[pl.ds(k * _L, _L)] = ones * 0

    pad_bb = jnp.where(wid == _NW - 1, 0, _NB - 1)
    for k in range(256 // _L):
        blist_v[pl.ds(k * _L, _L)] = iota * 0 + pad_bb

    def scan_chunk(c, hcnt):
        pltpu.sync_copy(idx_hbm.at[pl.ds(c * _CHUNK, _CHUNK)], chunk_v)

        def scan_vec(k, hcnt):
            jv = chunk_v[pl.ds(k * _L, _L)]
            bbv = lax.shift_right_logical(jv, 7)
            m = (bbv >= lo) & (bbv < hi)
            mi = m.astype(jnp.int32)
            rank = plsc.cumsum(mi) - mi
            slot = hcnt + rank
            plsc.store_scatter(hits_j, [slot], jv, mask=m)
            plsc.store_scatter(hits_p, [slot], c * _CHUNK + k * _L + iota,
                               mask=m)
            plsc.store_scatter(mark_v, [bbv - lo], ones, mask=m)
            return hcnt + jnp.sum(mi)

        return lax.fori_loop(0, _CHUNK // _L, scan_vec, hcnt)

    hcnt = lax.fori_loop(0, BATCH // _CHUNK, scan_chunk, 0)

    mcnt = 0
    for k in range(256 // _L):
        mk = mark_v[pl.ds(k * _L, _L)]
        m = mk > 0
        mi = m.astype(jnp.int32)
        rank = plsc.cumsum(mi) - mi
        plsc.store_scatter(blist_v, [mcnt + rank], iota + (k * _L + lo),
                           mask=m)
        mcnt = mcnt + jnp.sum(mi)

    nvec = lax.div(hcnt + _L - 1, _L)

    rings = [blk0_v, blk1_v, blk2_v, blk3_v]
    sems = [sem0, sem1, sem2, sem3]

    def fire(mth, k):
        bb = _vscalar(blist_v, mth)

        @pl.when(bb == _NB - 1)
        def _():
            pltpu.async_copy(tail_hbm, rings[k], sems[k])

        @pl.when(bb != _NB - 1)
        def _():
            off = pl.multiple_of(bb * 128, 128)
            pltpu.async_copy(table_hbm.at[:, pl.ds(off, 128)], rings[k],
                             sems[k])

    for k in range(4):
        fire(k, k)

    def rescan(bb, outcnt):
        def rescan_vec(v, nb):
            ids = v * _L + iota
            jv = plsc.load_gather(hits_j, [ids])
            pv = plsc.load_gather(hits_p, [ids])
            m = lax.shift_right_logical(jv, 7) == bb
            mi = m.astype(jnp.int32)
            rank = plsc.cumsum(mi) - mi
            slot = outcnt + nb + rank
            plsc.store_scatter(lane_v, [slot], jv & 127, mask=m)
            plsc.store_scatter(
                pos_v, [lax.shift_right_logical(slot, 7), slot & 127], pv,
                mask=m)
            return nb + jnp.sum(mi)

        return lax.fori_loop(0, nvec, rescan_vec, 0)

    def make_extract(blk):
        def extract(h, carry):
            lane = plsc.load_gather(
                lane_v, [jnp.full((_L,), 0, jnp.int32) + h])
            for g in range(H_DIM // _L):
                x = plsc.load_gather(blk, [iota + g * _L, lane])
                out_v[h, pl.ds(g * _L, _L)] = 1.0 / (1.0 + jnp.exp(-x))
            return carry
        return extract

    def do_round(r, outcnt):
        for k in range(4):
            mth = r * 4 + k
            bb = _vscalar(blist_v, mth)
            nb = rescan(bb, outcnt)
            pltpu.make_async_copy(
                table_hbm.at[:, pl.ds(0, 128)], rings[k], sems[k]).wait()
            lax.fori_loop(outcnt, outcnt + nb, make_extract(rings[k]), 0)
            fire(mth + 4, k)
            outcnt = outcnt + nb
        return outcnt

    nrounds = lax.div(mcnt + 3, 4)
    lax.fori_loop(0, nrounds, do_round, 0)

    for k in range(4):
        pltpu.make_async_copy(
            table_hbm.at[:, pl.ds(0, 128)], rings[k], sems[k]).wait()

    copies = []
    for q in range(_CAP // 128):
        copies.append(pltpu.async_copy(
            out_v.at[pl.ds(q * 128, 128)], out_hbm.at[pos_v.at[q]], sem_out))
    for c in copies:
        c.wait()


def kernel(X, Y, gate_theta):
    del X
    table_t = gate_theta.T
    tail = jnp.pad(gate_theta[(_NB - 1) * 128:].T,
                   ((0, 0), (0, _NB * 128 - NUM_E)))
    out2 = _gate_sc(table_t, tail, Y.astype(jnp.int32))
    return out2[:BATCH, :H_DIM]


# speedup vs baseline: 2.1945x; 1.1811x over previous
import functools

import jax
import jax.numpy as jnp
from jax import lax
from jax.experimental import pallas as pl
from jax.experimental.pallas import tpu as pltpu
from jax.experimental.pallas import tpu_sc as plsc

NUM_E = 1000000
H_DIM = 64
BATCH = 16384

_NC = 2
_NS = 16
_L = 16
_NW = _NC * _NS
_NB = (NUM_E + 127) // 128
_BPW = (_NB + _NW - 1) // _NW
_CAP = 640
_CHUNK = 2048
_TRASH = BATCH

_mesh = plsc.VectorSubcoreMesh(core_axis_name="c", subcore_axis_name="s")


def _vscalar(ref, i):
    v = plsc.load_gather(ref, [jnp.full((_L,), 0, jnp.int32) + i])
    return jax.lax.reduce_max(v, (0,))


@functools.partial(
    pl.kernel,
    mesh=_mesh,
    out_type=jax.ShapeDtypeStruct((BATCH + 1, 2 * H_DIM), jnp.float32),
    scratch_types=[
        pltpu.VMEM((_CHUNK,), jnp.int32),
        pltpu.VMEM((_CAP,), jnp.int32),
        pltpu.VMEM((_CAP,), jnp.int32),
        pltpu.VMEM((_CAP,), jnp.int32),
        pltpu.VMEM((_CAP // 128, 128), jnp.int32),
        pltpu.VMEM((256,), jnp.int32),
        pltpu.VMEM((256,), jnp.int32),
        pltpu.VMEM((H_DIM, 128), jnp.float32),
        pltpu.VMEM((H_DIM, 128), jnp.float32),
        pltpu.VMEM((H_DIM, 128), jnp.float32),
        pltpu.VMEM((H_DIM, 128), jnp.float32),
        pltpu.VMEM((_CAP, 2 * H_DIM), jnp.float32),
        pltpu.VMEM((256,), jnp.int32),
        pltpu.VMEM((256,), jnp.int32),
        pltpu.SemaphoreType.DMA,
        pltpu.SemaphoreType.DMA,
        pltpu.SemaphoreType.DMA,
        pltpu.SemaphoreType.DMA,
        pltpu.SemaphoreType.DMA,
    ],
    compiler_params=pltpu.CompilerParams(needs_layout_passes=False),
)
def _gate_sc(table_hbm, tail_hbm, idx_hbm, out_hbm,
             chunk_v, hits_j, hits_p, lane_v, pos_v, mark_v, blist_v,
             blk0_v, blk1_v, blk2_v, blk3_v, out_v,
             cnt_v, start_v,
             sem0, sem1, sem2, sem3, sem_out):
    wid = lax.axis_index("s") * _NC + lax.axis_index("c")
    lo = wid * _BPW
    hi = jnp.minimum(lo + _BPW, _NB)
    iota = lax.iota(jnp.int32, _L)
    ones = jnp.full((_L,), 1, jnp.int32)

    for k in range(_CAP // _L):
        hits_j[pl.ds(k * _L, _L)] = ones * -1
    for q in range(_CAP // 128):
        for g in range(128 // _L):
            pos_v[q, pl.ds(g * _L, _L)] = ones * _TRASH
    for k in range(256 // _L):
        mark_v[pl.ds(k * _L, _L)] = ones * 0

    pad_bb = jnp.where(wid == _NW - 1, 0, _NB - 1)
    for k in range(256 // _L):
        blist_v[pl.ds(k * _L, _L)] = iota * 0 + pad_bb

    def scan_chunk(c, hcnt):
        pltpu.sync_copy(idx_hbm.at[pl.ds(c * _CHUNK, _CHUNK)], chunk_v)

        def scan_vec(k, hcnt):
            jv = chunk_v[pl.ds(k * _L, _L)]
            bbv = lax.shift_right_logical(jv, 7)
            m = (bbv >= lo) & (bbv < hi)
            mi = m.astype(jnp.int32)
            rank = plsc.cumsum(mi) - mi
            slot = hcnt + rank
            plsc.store_scatter(hits_j, [slot], jv, mask=m)
            plsc.store_scatter(hits_p, [slot], c * _CHUNK + k * _L + iota,
                               mask=m)
            plsc.store_scatter(mark_v, [bbv - lo], ones, mask=m)
            return hcnt + jnp.sum(mi)

        return lax.fori_loop(0, _CHUNK // _L, scan_vec, hcnt)

    hcnt = lax.fori_loop(0, BATCH // _CHUNK, scan_chunk, 0)

    mcnt = 0
    for k in range(256 // _L):
        mk = mark_v[pl.ds(k * _L, _L)]
        m = mk > 0
        mi = m.astype(jnp.int32)
        rank = plsc.cumsum(mi) - mi
        plsc.store_scatter(blist_v, [mcnt + rank], iota + (k * _L + lo),
                           mask=m)
        mcnt = mcnt + jnp.sum(mi)

    zero16 = iota * 0

    def count_hit(h, cnts):
        jv = plsc.load_gather(hits_j, [zero16 + h])
        bbl = lax.shift_right_logical(jv, 7) - lo
        grp = lax.shift_right_logical(bbl, 4)
        oh = (iota == (bbl & 15)).astype(jnp.int32)
        return tuple(
            cnts[q] + jnp.where(grp == q, oh, zero16) for q in range(16))

    cnts = lax.fori_loop(0, hcnt, count_hit, (zero16,) * 16)

    starts = []
    run = 0
    for q in range(16):
        starts.append(run + plsc.cumsum(cnts[q]) - cnts[q])
        run = run + jnp.sum(cnts[q])
    for q in range(16):
        cnt_v[pl.ds(q * _L, _L)] = cnts[q]
        start_v[pl.ds(q * _L, _L)] = starts[q]

    def place(h, offs):
        jv = plsc.load_gather(hits_j, [zero16 + h])
        pv = plsc.load_gather(hits_p, [zero16 + h])
        bbl = lax.shift_right_logical(jv, 7) - lo
        grp = lax.shift_right_logical(bbl, 4)
        oh = (iota == (bbl & 15)).astype(jnp.int32)
        slotv = zero16
        for q in range(16):
            slotv = slotv + jnp.where(grp == q, offs[q], zero16)
        slot = zero16 + jnp.sum(slotv * oh)
        lane0 = iota == 0
        plsc.store_scatter(lane_v, [slot], jv & 127, mask=lane0)
        plsc.store_scatter(
            pos_v, [lax.shift_right_logical(slot, 7), slot & 127], pv,
            mask=lane0)
        return tuple(
            offs[q] + jnp.where(grp == q, oh, zero16) for q in range(16))

    lax.fori_loop(0, hcnt, place, tuple(starts))

    rings = [blk0_v, blk1_v, blk2_v, blk3_v]
    sems = [sem0, sem1, sem2, sem3]

    def fire(mth, k):
        bb = _vscalar(blist_v, mth)

        @pl.when(bb == _NB - 1)
        def _():
            pltpu.async_copy(tail_hbm, rings[k], sems[k])

        @pl.when(bb != _NB - 1)
        def _():
            off = pl.multiple_of(bb * 128, 128)
            pltpu.async_copy(table_hbm.at[:, pl.ds(off, 128)], rings[k],
                             sems[k])

    for k in range(4):
        fire(k, k)

    def make_extract(blk):
        def extract(h, carry):
            lane = plsc.load_gather(
                lane_v, [jnp.full((_L,), 0, jnp.int32) + h])
            for g in range(H_DIM // _L):
                x = plsc.load_gather(blk, [iota + g * _L, lane])
                out_v[h, pl.ds(g * _L, _L)] = 1.0 / (1.0 + jnp.exp(-x))
            return carry
        return extract

    def do_round(r, carry):
        for k in range(4):
            mth = r * 4 + k
            bb = _vscalar(blist_v, mth)
            valid = bb != pad_bb
            bbl = jnp.clip(bb - lo, 0, 255)
            start = jnp.where(valid, _vscalar(start_v, bbl), 0)
            end = start + jnp.where(valid, _vscalar(cnt_v, bbl), 0)
            pltpu.make_async_copy(
                table_hbm.at[:, pl.ds(0, 128)], rings[k], sems[k]).wait()
            lax.fori_loop(start, end, make_extract(rings[k]), 0)
            fire(mth + 4, k)
        return carry

    nrounds = lax.div(mcnt + 3, 4)
    lax.fori_loop(0, nrounds, do_round, 0)

    for k in range(4):
        pltpu.make_async_copy(
            table_hbm.at[:, pl.ds(0, 128)], rings[k], sems[k]).wait()

    copies = []
    for q in range(_CAP // 128):
        copies.append(pltpu.async_copy(
            out_v.at[pl.ds(q * 128, 128)], out_hbm.at[pos_v.at[q]], sem_out))
    for c in copies:
        c.wait()


def kernel(X, Y, gate_theta):
    del X
    table_t = gate_theta.T
    tail = jnp.pad(gate_theta[(_NB - 1) * 128:].T,
                   ((0, 0), (0, _NB * 128 - NUM_E)))
    out2 = _gate_sc(table_t, tail, Y.astype(jnp.int32))
    return out2[:BATCH, :H_DIM]
